# baseline (speedup 1.0000x reference)
import math

import jax
import jax.numpy as jnp
from jax import lax
from jax.experimental import pallas as pl
from jax.experimental.pallas import tpu as pltpu
from jax.experimental.pallas import tpu_sc as plsc

B = 1024
L = 200
D = 128
N = B * L
NC = 2
NS = 16
NW = NC * NS
PER_W = N // NW
C = 128
NCHUNK = PER_W // C
K = 5
LA = 3
NGRP = NCHUNK // K
LANES = 16
SCALE = math.sqrt(float(D))


def _body(ids_hbm, tok_hbm, pos_hbm, out_hbm, idxall, rows, pos2, sg, so):
    wid = lax.axis_index("s") * NC + lax.axis_index("c")
    base_w = wid * PER_W

    pltpu.sync_copy(ids_hbm.at[wid], idxall)
    pltpu.sync_copy(pos_hbm.at[pl.ds(0, L)], pos2)

    for b in range(LA):
        pltpu.async_copy(tok_hbm.at[idxall.at[b]], rows.at[b], sg[b])

    LAG = K - LA

    @pl.loop(0, NGRP)
    def _grp(g):
        for b in range(K):
            c = g * K + b
            bn = (b + LA) % K

            @pl.when(c >= LAG)
            def _drain():
                pbase = base_w + (c - LAG) * C
                pltpu.make_async_copy(
                    rows.at[bn], out_hbm.at[pl.ds(pbase, C)], so[bn]
                ).wait()

            @pl.when(c + LA < NCHUNK)
            def _fire():
                pltpu.async_copy(
                    tok_hbm.at[idxall.at[c + LA]], rows.at[bn], sg[bn]
                )

            pltpu.make_async_copy(
                tok_hbm.at[idxall.at[c]], rows.at[b], sg[b]
            ).wait()
            base = base_w + c * C
            p0 = lax.rem(base, L)
            s = jnp.minimum(L - p0, C)

            @plsc.parallel_loop(0, s, unroll=4)
            def _row_lo(j):
                for d in range(D // LANES):
                    sl = pl.ds(d * LANES, LANES)
                    rows[b, j, sl] = rows[b, j, sl] * SCALE + pos2[p0 + j, sl]

            @plsc.parallel_loop(s, C, unroll=4)
            def _row_hi(j):
                for d in range(D // LANES):
                    sl = pl.ds(d * LANES, LANES)
                    rows[b, j, sl] = rows[b, j, sl] * SCALE + pos2[j - s, sl]

            pltpu.async_copy(rows.at[b], out_hbm.at[pl.ds(base, C)], so[b])

    for t in range(LAG):
        c = NCHUNK - LAG + t
        pbase = base_w + c * C
        pltpu.make_async_copy(
            rows.at[c % K], out_hbm.at[pl.ds(pbase, C)], so[c % K]
        ).wait()


@jax.jit
def _run(ids2d, tok_table, pos_table):
    mesh = plsc.VectorSubcoreMesh(core_axis_name="c", subcore_axis_name="s")
    f = pl.kernel(
        _body,
        out_type=jax.ShapeDtypeStruct((N, D), jnp.float32),
        mesh=mesh,
        scratch_types=[
            pltpu.VMEM((NCHUNK, C), jnp.int32),
            pltpu.VMEM((K, C, D), jnp.float32),
            pltpu.VMEM((L, D), jnp.float32),
            [pltpu.SemaphoreType.DMA] * K,
            [pltpu.SemaphoreType.DMA] * K,
        ],
    )
    return f(ids2d, tok_table, pos_table)


def kernel(input_ids, tok_table, pos_table):
    ids2d = input_ids.reshape(NW, NCHUNK, C).astype(jnp.int32)
    out = _run(ids2d, tok_table, pos_table)
    return out.reshape(B, L, D)


# speedup vs baseline: 1.0072x; 1.0072x over previous
import math

import jax
import jax.numpy as jnp
from jax import lax
from jax.experimental import pallas as pl
from jax.experimental.pallas import tpu as pltpu
from jax.experimental.pallas import tpu_sc as plsc

B = 1024
L = 200
D = 128
N = B * L
NC = 2
NS = 16
NW = NC * NS
PER_W = N // NW
C = 64
NCHUNK = PER_W // C
K = 10
LA = 7
NGRP = NCHUNK // K
LANES = 16
PE = L + C + 8
SCALE = math.sqrt(float(D))


def _body(ids_hbm, tok_hbm, pos_hbm, out_hbm, idxall, rows, pos2, sg, so):
    wid = lax.axis_index("s") * NC + lax.axis_index("c")
    base_w = wid * PER_W

    pltpu.sync_copy(ids_hbm.at[wid], idxall)
    pltpu.sync_copy(pos_hbm.at[pl.ds(0, L)], pos2.at[pl.ds(0, L)])
    pltpu.sync_copy(pos_hbm.at[pl.ds(0, C + 8)], pos2.at[pl.ds(L, C + 8)])

    for b in range(LA):
        pltpu.async_copy(tok_hbm.at[idxall.at[b]], rows.at[b], sg[b])

    LAG = K - LA

    @pl.loop(0, NGRP)
    def _grp(g):
        for b in range(K):
            c = g * K + b
            bn = (b + LA) % K

            @pl.when(c >= LAG)
            def _drain():
                pbase = base_w + (c - LAG) * C
                pltpu.make_async_copy(
                    rows.at[bn], out_hbm.at[pl.ds(pbase, C)], so[bn]
                ).wait()

            @pl.when(c + LA < NCHUNK)
            def _fire():
                pltpu.async_copy(
                    tok_hbm.at[idxall.at[c + LA]], rows.at[bn], sg[bn]
                )

            pltpu.make_async_copy(
                tok_hbm.at[idxall.at[c]], rows.at[b], sg[b]
            ).wait()
            base = base_w + c * C
            p0 = lax.rem(base, L)

            @plsc.parallel_loop(0, C, unroll=4)
            def _row(j):
                for d in range(D // LANES):
                    sl = pl.ds(d * LANES, LANES)
                    rows[b, j, sl] = rows[b, j, sl] * SCALE + pos2[p0 + j, sl]

            pltpu.async_copy(rows.at[b], out_hbm.at[pl.ds(base, C)], so[b])

    for t in range(LAG):
        c = NCHUNK - LAG + t
        pbase = base_w + c * C
        pltpu.make_async_copy(
            rows.at[c % K], out_hbm.at[pl.ds(pbase, C)], so[c % K]
        ).wait()


@jax.jit
def _run(ids2d, tok_table, pos_table):
    mesh = plsc.VectorSubcoreMesh(core_axis_name="c", subcore_axis_name="s")
    f = pl.kernel(
        _body,
        out_type=jax.ShapeDtypeStruct((N, D), jnp.float32),
        mesh=mesh,
        scratch_types=[
            pltpu.VMEM((NCHUNK, C), jnp.int32),
            pltpu.VMEM((K, C, D), jnp.float32),
            pltpu.VMEM((PE, D), jnp.float32),
            [pltpu.SemaphoreType.DMA] * K,
            [pltpu.SemaphoreType.DMA] * K,
        ],
    )
    return f(ids2d, tok_table, pos_table)


def kernel(input_ids, tok_table, pos_table):
    ids2d = input_ids.reshape(NW, NCHUNK, C).astype(jnp.int32)
    out = _run(ids2d, tok_table, pos_table)
    return out.reshape(B, L, D)


# speedup vs baseline: 1.0094x; 1.0021x over previous
import math

import jax
import jax.numpy as jnp
from jax import lax
from jax.experimental import pallas as pl
from jax.experimental.pallas import tpu as pltpu
from jax.experimental.pallas import tpu_sc as plsc

B = 1024
L = 200
D = 128
N = B * L
NC = 2
NS = 16
NW = NC * NS
PER_W = N // NW
C = 64
NCHUNK = PER_W // C
K = 10
LA = 6
NGRP = NCHUNK // K
LANES = 16
PE = L + C + 8
SCALE = math.sqrt(float(D))


def _body(ids_hbm, tok_hbm, pos_hbm, out_hbm, idxall, rows, pos2, sg, so):
    wid = lax.axis_index("s") * NC + lax.axis_index("c")
    base_w = wid * PER_W

    pltpu.sync_copy(ids_hbm.at[wid], idxall)
    pltpu.sync_copy(pos_hbm.at[pl.ds(0, L)], pos2.at[pl.ds(0, L)])
    pltpu.sync_copy(pos_hbm.at[pl.ds(0, C + 8)], pos2.at[pl.ds(L, C + 8)])

    for b in range(LA):
        pltpu.async_copy(tok_hbm.at[idxall.at[b]], rows.at[b], sg[b])

    LAG = K - LA

    @pl.loop(0, NGRP)
    def _grp(g):
        for b in range(K):
            c = g * K + b
            bn = (b + LA) % K

            @pl.when(c >= LAG)
            def _drain():
                pbase = base_w + (c - LAG) * C
                pltpu.make_async_copy(
                    rows.at[bn], out_hbm.at[pl.ds(pbase, C)], so[bn]
                ).wait()

            @pl.when(c + LA < NCHUNK)
            def _fire():
                pltpu.async_copy(
                    tok_hbm.at[idxall.at[c + LA]], rows.at[bn], sg[bn]
                )

            pltpu.make_async_copy(
                tok_hbm.at[idxall.at[c]], rows.at[b], sg[b]
            ).wait()
            base = base_w + c * C
            p0 = lax.rem(base, L)

            @plsc.parallel_loop(0, C, unroll=4)
            def _row(j):
                for d in range(D // LANES):
                    sl = pl.ds(d * LANES, LANES)
                    rows[b, j, sl] = rows[b, j, sl] * SCALE + pos2[p0 + j, sl]

            pltpu.async_copy(rows.at[b], out_hbm.at[pl.ds(base, C)], so[b])

    for t in range(LAG):
        c = NCHUNK - LAG + t
        pbase = base_w + c * C
        pltpu.make_async_copy(
            rows.at[c % K], out_hbm.at[pl.ds(pbase, C)], so[c % K]
        ).wait()


@jax.jit
def _run(ids2d, tok_table, pos_table):
    mesh = plsc.VectorSubcoreMesh(core_axis_name="c", subcore_axis_name="s")
    f = pl.kernel(
        _body,
        out_type=jax.ShapeDtypeStruct((N, D), jnp.float32),
        mesh=mesh,
        scratch_types=[
            pltpu.VMEM((NCHUNK, C), jnp.int32),
            pltpu.VMEM((K, C, D), jnp.float32),
            pltpu.VMEM((PE, D), jnp.float32),
            [pltpu.SemaphoreType.DMA] * K,
            [pltpu.SemaphoreType.DMA] * K,
        ],
    )
    return f(ids2d, tok_table, pos_table)


def kernel(input_ids, tok_table, pos_table):
    ids2d = input_ids.reshape(NW, NCHUNK, C).astype(jnp.int32)
    out = _run(ids2d, tok_table, pos_table)
    return out.reshape(B, L, D)


# speedup vs baseline: 1.0193x; 1.0099x over previous
import math

import jax
import jax.numpy as jnp
from jax import lax
from jax.experimental import pallas as pl
from jax.experimental.pallas import tpu as pltpu
from jax.experimental.pallas import tpu_sc as plsc

B = 1024
L = 200
D = 128
N = B * L
NC = 2
NS = 16
NW = NC * NS
PER_W = N // NW
C = 64
NCHUNK = PER_W // C
K = 10
LA = 5
NGRP = NCHUNK // K
LANES = 16
PE = L + C + 8
SCALE = math.sqrt(float(D))


def _body(ids_hbm, tok_hbm, pos_hbm, out_hbm, idxall, rows, pos2, sg, so):
    wid = lax.axis_index("s") * NC + lax.axis_index("c")
    base_w = wid * PER_W

    pltpu.sync_copy(ids_hbm.at[wid], idxall)
    pltpu.sync_copy(pos_hbm.at[pl.ds(0, L)], pos2.at[pl.ds(0, L)])
    pltpu.sync_copy(pos_hbm.at[pl.ds(0, C + 8)], pos2.at[pl.ds(L, C + 8)])

    for b in range(LA):
        pltpu.async_copy(tok_hbm.at[idxall.at[b]], rows.at[b], sg[b])

    LAG = K - LA

    @pl.loop(0, NGRP)
    def _grp(g):
        for b in range(K):
            c = g * K + b
            bn = (b + LA) % K

            @pl.when(c >= LAG)
            def _drain():
                pbase = base_w + (c - LAG) * C
                pltpu.make_async_copy(
                    rows.at[bn], out_hbm.at[pl.ds(pbase, C)], so[bn]
                ).wait()

            @pl.when(c + LA < NCHUNK)
            def _fire():
                pltpu.async_copy(
                    tok_hbm.at[idxall.at[c + LA]], rows.at[bn], sg[bn]
                )

            pltpu.make_async_copy(
                tok_hbm.at[idxall.at[c]], rows.at[b], sg[b]
            ).wait()
            base = base_w + c * C
            p0 = lax.rem(base, L)

            @plsc.parallel_loop(0, C, unroll=4)
            def _row(j):
                for d in range(D // LANES):
                    sl = pl.ds(d * LANES, LANES)
                    rows[b, j, sl] = rows[b, j, sl] * SCALE + pos2[p0 + j, sl]

            pltpu.async_copy(rows.at[b], out_hbm.at[pl.ds(base, C)], so[b])

    for t in range(LAG):
        c = NCHUNK - LAG + t
        pbase = base_w + c * C
        pltpu.make_async_copy(
            rows.at[c % K], out_hbm.at[pl.ds(pbase, C)], so[c % K]
        ).wait()


@jax.jit
def _run(ids2d, tok_table, pos_table):
    mesh = plsc.VectorSubcoreMesh(core_axis_name="c", subcore_axis_name="s")
    f = pl.kernel(
        _body,
        out_type=jax.ShapeDtypeStruct((N, D), jnp.float32),
        mesh=mesh,
        scratch_types=[
            pltpu.VMEM((NCHUNK, C), jnp.int32),
            pltpu.VMEM((K, C, D), jnp.float32),
            pltpu.VMEM((PE, D), jnp.float32),
            [pltpu.SemaphoreType.DMA] * K,
            [pltpu.SemaphoreType.DMA] * K,
        ],
    )
    return f(ids2d, tok_table, pos_table)


def kernel(input_ids, tok_table, pos_table):
    ids2d = input_ids.reshape(NW, NCHUNK, C).astype(jnp.int32)
    out = _run(ids2d, tok_table, pos_table)
    return out.reshape(B, L, D)
